# Initial kernel scaffold; baseline (speedup 1.0000x reference)
import jax
import jax.numpy as jnp
from jax import lax
from jax.experimental import pallas as pl
from jax.experimental.pallas import tpu as pltpu
from jax.experimental.pallas import tpu_sc as plsc

N_COINS = 100000
EMBED_DIM = 16
BATCH = 16384
HIST = 50
B_TOTAL = BATCH * HIST

NC = 2
NS = 16
NW = NC * NS
B_PER_W = B_TOTAL // NW
CHUNK = 3200
N_CHUNKS = B_PER_W // CHUNK


def _emb_body(idx_hbm, table_hbm, out_hbm, idx_v, rows_v, sem):
    wid = lax.axis_index("s") * NC + lax.axis_index("c")
    base = wid * B_PER_W

    def body(i, carry):
        off = base + i * CHUNK
        pltpu.sync_copy(idx_hbm.at[pl.ds(off, CHUNK)], idx_v)
        pltpu.async_copy(table_hbm.at[idx_v], rows_v, sem).wait()
        pltpu.sync_copy(rows_v, out_hbm.at[pl.ds(off, CHUNK)])
        return carry

    lax.fori_loop(0, N_CHUNKS, body, 0)


def kernel(coin_id, table):
    idx = coin_id.reshape(-1).astype(jnp.int32)
    mesh = plsc.VectorSubcoreMesh(core_axis_name="c", subcore_axis_name="s")
    k = pl.kernel(
        _emb_body,
        mesh=mesh,
        out_type=jax.ShapeDtypeStruct((B_TOTAL, EMBED_DIM), jnp.float32),
        scratch_types=[
            pltpu.VMEM((CHUNK,), jnp.int32),
            pltpu.VMEM((CHUNK, EMBED_DIM), jnp.float32),
            pltpu.SemaphoreType.DMA,
        ],
    )
    out = k(idx, table)
    return out.reshape(BATCH, HIST, EMBED_DIM)


# speedup vs baseline: 3.9095x; 3.9095x over previous
import jax
import jax.numpy as jnp
from jax import lax
from jax.experimental import pallas as pl
from jax.experimental.pallas import tpu as pltpu
from jax.experimental.pallas import tpu_sc as plsc

N_COINS = 100000
EMBED_DIM = 16
BATCH = 16384
HIST = 50
B_TOTAL = BATCH * HIST

NC = 2
NS = 16
NW = NC * NS
B_PER_W = B_TOTAL // NW
CHUNK = 3200
N_CHUNKS = B_PER_W // CHUNK


def _emb_body(idx_hbm, table_hbm, out_hbm, idx_v, rows_v, sem):
    wid = lax.axis_index("s") * NC + lax.axis_index("c")
    base = wid * B_PER_W

    def body(i, carry):
        off = base + i * CHUNK
        pltpu.sync_copy(idx_hbm.at[pl.ds(off, CHUNK)], idx_v)
        pltpu.async_copy(table_hbm.at[idx_v], rows_v, sem).wait()
        pltpu.sync_copy(rows_v, out_hbm.at[pl.ds(off, CHUNK)])
        return carry

    lax.fori_loop(0, N_CHUNKS, body, 0)


def kernel(coin_id, table):
    idx = coin_id.reshape(-1).astype(jnp.int32)
    mesh = plsc.VectorSubcoreMesh(core_axis_name="c", subcore_axis_name="s")
    k = pl.kernel(
        _emb_body,
        mesh=mesh,
        out_type=jax.ShapeDtypeStruct((B_TOTAL, EMBED_DIM), jnp.float32),
        scratch_types=[
            pltpu.VMEM((CHUNK,), jnp.int32),
            pltpu.VMEM((CHUNK, EMBED_DIM), jnp.float32),
            pltpu.SemaphoreType.DMA,
        ],
        compiler_params=pltpu.CompilerParams(use_tc_tiling_on_sc=False),
    )
    out = k(idx, table)
    return out.reshape(BATCH, HIST, EMBED_DIM)


# speedup vs baseline: 3.9476x; 1.0097x over previous
import jax
import jax.numpy as jnp
from jax import lax
from jax.experimental import pallas as pl
from jax.experimental.pallas import tpu as pltpu
from jax.experimental.pallas import tpu_sc as plsc

N_COINS = 100000
EMBED_DIM = 16
BATCH = 16384
HIST = 50
B_TOTAL = BATCH * HIST

NC = 2
NS = 16
NW = NC * NS
B_PER_W = B_TOTAL // NW
CHUNK = 3200
N_CHUNKS = B_PER_W // CHUNK


def _emb_body(idx_hbm, table_hbm, out_hbm, idx_v, rows0, rows1, gsem, ssem):
    wid = lax.axis_index("s") * NC + lax.axis_index("c")
    base = wid * B_PER_W
    pltpu.sync_copy(idx_hbm.at[pl.ds(base, B_PER_W)], idx_v)
    rows = (rows0, rows1)
    stores = [None, None]
    for i in range(N_CHUNKS):
        b = i % 2
        if stores[b] is not None:
            stores[b].wait()
        g = pltpu.async_copy(
            table_hbm.at[idx_v.at[pl.ds(i * CHUNK, CHUNK)]], rows[b], gsem)
        g.wait()
        stores[b] = pltpu.async_copy(
            rows[b], out_hbm.at[pl.ds(base + i * CHUNK, CHUNK)], ssem)
    for s in stores:
        s.wait()


def kernel(coin_id, table):
    idx = coin_id.reshape(-1).astype(jnp.int32)
    mesh = plsc.VectorSubcoreMesh(core_axis_name="c", subcore_axis_name="s")
    k = pl.kernel(
        _emb_body,
        mesh=mesh,
        out_type=jax.ShapeDtypeStruct((B_TOTAL, EMBED_DIM), jnp.float32),
        scratch_types=[
            pltpu.VMEM((B_PER_W,), jnp.int32),
            pltpu.VMEM((CHUNK, EMBED_DIM), jnp.float32),
            pltpu.VMEM((CHUNK, EMBED_DIM), jnp.float32),
            pltpu.SemaphoreType.DMA,
            pltpu.SemaphoreType.DMA,
        ],
        compiler_params=pltpu.CompilerParams(use_tc_tiling_on_sc=False),
    )
    out = k(idx, table)
    return out.reshape(BATCH, HIST, EMBED_DIM)


# speedup vs baseline: 3.9691x; 1.0054x over previous
import jax
import jax.numpy as jnp
from jax import lax
from jax.experimental import pallas as pl
from jax.experimental.pallas import tpu as pltpu
from jax.experimental.pallas import tpu_sc as plsc

N_COINS = 100000
EMBED_DIM = 16
BATCH = 16384
HIST = 50
B_TOTAL = BATCH * HIST

NC = 2
NS = 16
NW = NC * NS
B_PER_W = B_TOTAL // NW
CHUNK = 1600
N_CHUNKS = B_PER_W // CHUNK
NBUF = 4
LOOKAHEAD = NBUF - 1


def _emb_body(idx_hbm, table_hbm, out_hbm, idx_v, *bufs_and_sems):
    rows = bufs_and_sems[:NBUF]
    gsems = bufs_and_sems[NBUF:2 * NBUF]
    ssems = bufs_and_sems[2 * NBUF:3 * NBUF]
    wid = lax.axis_index("s") * NC + lax.axis_index("c")
    base = wid * B_PER_W
    pltpu.sync_copy(idx_hbm.at[pl.ds(base, B_PER_W)], idx_v)

    def gather(j):
        b = j % NBUF
        return pltpu.async_copy(
            table_hbm.at[idx_v.at[pl.ds(j * CHUNK, CHUNK)]], rows[b], gsems[b])

    gds = [None] * NBUF
    sds = [None] * NBUF
    for j in range(LOOKAHEAD):
        gds[j % NBUF] = gather(j)
    for i in range(N_CHUNKS):
        b = i % NBUF
        j = i + LOOKAHEAD
        if j < N_CHUNKS:
            bj = j % NBUF
            if sds[bj] is not None:
                sds[bj].wait()
            gds[bj] = gather(j)
        gds[b].wait()
        sds[b] = pltpu.async_copy(
            rows[b], out_hbm.at[pl.ds(base + i * CHUNK, CHUNK)], ssems[b])
    for s in sds:
        s.wait()


def kernel(coin_id, table):
    idx = coin_id.reshape(-1).astype(jnp.int32)
    mesh = plsc.VectorSubcoreMesh(core_axis_name="c", subcore_axis_name="s")
    k = pl.kernel(
        _emb_body,
        mesh=mesh,
        out_type=jax.ShapeDtypeStruct((B_TOTAL, EMBED_DIM), jnp.float32),
        scratch_types=(
            [pltpu.VMEM((B_PER_W,), jnp.int32)]
            + [pltpu.VMEM((CHUNK, EMBED_DIM), jnp.float32)] * NBUF
            + [pltpu.SemaphoreType.DMA] * (2 * NBUF)
        ),
        compiler_params=pltpu.CompilerParams(use_tc_tiling_on_sc=False),
    )
    out = k(idx, table)
    return out.reshape(BATCH, HIST, EMBED_DIM)


# speedup vs baseline: 3.9833x; 1.0036x over previous
import jax
import jax.numpy as jnp
from jax import lax
from jax.experimental import pallas as pl
from jax.experimental.pallas import tpu as pltpu
from jax.experimental.pallas import tpu_sc as plsc

N_COINS = 100000
EMBED_DIM = 16
BATCH = 16384
HIST = 50
B_TOTAL = BATCH * HIST

NC = 2
NS = 16
NW = NC * NS
B_PER_W = B_TOTAL // NW
CHUNK = 800
N_CHUNKS = B_PER_W // CHUNK

ROWS_PER_TILE = N_COINS // NS


def _emb_body(idx_hbm, table_hbm, out_hbm, shared_tab,
              idx0, idx1, rows0, rows1,
              isem0, isem1, gsem0, gsem1, ssem0, ssem1):
    sid = lax.axis_index("s")
    wid = sid * NC + lax.axis_index("c")
    base = wid * B_PER_W
    idxs = (idx0, idx1)
    rows = (rows0, rows1)
    isems = (isem0, isem1)
    gsems = (gsem0, gsem1)
    ssems = (ssem0, ssem1)

    tile_off = sid * ROWS_PER_TILE
    done = 0
    while done < ROWS_PER_TILE:
        n = min(CHUNK, ROWS_PER_TILE - done)
        buf = rows0 if n == CHUNK else rows0.at[pl.ds(0, n)]
        pltpu.sync_copy(table_hbm.at[pl.ds(tile_off + done, n)], buf)
        pltpu.sync_copy(buf, shared_tab.at[pl.ds(tile_off + done, n)])
        done += n
    plsc.subcore_barrier()

    def idx_load(j):
        b = j % 2
        return pltpu.async_copy(
            idx_hbm.at[pl.ds(base + j * CHUNK, CHUNK)], idxs[b], isems[b])

    ids = [None, None]
    sds = [None, None]
    ids[0] = idx_load(0)
    for i in range(N_CHUNKS):
        b = i % 2
        if i + 1 < N_CHUNKS:
            ids[1 - b] = idx_load(i + 1)
        ids[b].wait()
        if sds[b] is not None:
            sds[b].wait()
        pltpu.async_copy(shared_tab.at[idxs[b]], rows[b], gsems[b]).wait()
        sds[b] = pltpu.async_copy(
            rows[b], out_hbm.at[pl.ds(base + i * CHUNK, CHUNK)], ssems[b])
    for s in sds:
        s.wait()


def kernel(coin_id, table):
    idx = coin_id.reshape(-1).astype(jnp.int32)
    mesh = plsc.VectorSubcoreMesh(core_axis_name="c", subcore_axis_name="s")
    k = pl.kernel(
        _emb_body,
        mesh=mesh,
        out_type=jax.ShapeDtypeStruct((B_TOTAL, EMBED_DIM), jnp.float32),
        scratch_types=(
            [pltpu.VMEM_SHARED((N_COINS, EMBED_DIM), jnp.float32)]
            + [pltpu.VMEM((CHUNK,), jnp.int32)] * 2
            + [pltpu.VMEM((CHUNK, EMBED_DIM), jnp.float32)] * 2
            + [pltpu.SemaphoreType.DMA] * 6
        ),
        compiler_params=pltpu.CompilerParams(use_tc_tiling_on_sc=False),
    )
    out = k(idx, table)
    return out.reshape(BATCH, HIST, EMBED_DIM)
